# baseline (speedup 1.0000x reference)
import jax
import jax.numpy as jnp
from jax import lax
from jax.experimental import pallas as pl
from jax.experimental.pallas import tpu as pltpu
from jax.experimental.pallas import tpu_sc as plsc

VOCAB_ROWS = 1_000_000
EMB_DIM = 32
NUM_TERMS = 50
NUM_BAGS = 16384
NUM_CORES = 2
NUM_SUBCORES = 16
NUM_WORKERS = NUM_CORES * NUM_SUBCORES
BAGS_PER_WORKER = NUM_BAGS // NUM_WORKERS
CHUNK = 128
CHUNKS_PER_TERM = BAGS_PER_WORKER // CHUNK


NBUF = 8
NUM_CHUNKS = NUM_TERMS * CHUNKS_PER_TERM


def _accumulate(acc, buf, c):
    for r in range(CHUNK):
        b = c * CHUNK + r
        for h in (0, 16):
            plsc.addupdate(acc.at[b, pl.ds(h, 16)], buf[r, pl.ds(h, 16)])


def _bag_sum_body(idx_hbm, weight_hbm, out_hbm, idx_v, acc, *ring):
    bufs = ring[:NBUF]
    sems = ring[NBUF:]
    wid = lax.axis_index("s") * NUM_CORES + lax.axis_index("c")
    base = wid * BAGS_PER_WORKER

    pltpu.sync_copy(idx_hbm.at[:, pl.ds(wid * CHUNKS_PER_TERM,
                                        CHUNKS_PER_TERM)], idx_v)

    zero = jnp.zeros((16,), jnp.float32)

    @pl.loop(0, BAGS_PER_WORKER)
    def _(i):
        acc[i, pl.ds(0, 16)] = zero
        acc[i, pl.ds(16, 16)] = zero

    def start(l, c, b):
        pltpu.async_copy(weight_hbm.at[idx_v.at[l, c]], bufs[b], sems[b])

    def wait(b):
        pltpu.make_async_copy(weight_hbm.at[pl.ds(0, CHUNK)], bufs[b],
                              sems[b]).wait()

    for b in range(NBUF):
        start(b // CHUNKS_PER_TERM, b % CHUNKS_PER_TERM, b)

    @pl.loop(0, NUM_CHUNKS, step=NBUF)
    def _(j):
        for b in range(NBUF):
            jj = j + b
            wait(b)
            _accumulate(acc, bufs[b], b % CHUNKS_PER_TERM)
            nxt = jj + NBUF

            @pl.when(nxt < NUM_CHUNKS)
            def _():
                start(nxt // CHUNKS_PER_TERM, b % CHUNKS_PER_TERM, b)

    pltpu.sync_copy(acc, out_hbm.at[pl.ds(base, BAGS_PER_WORKER)])


def kernel(input, weight):
    idx = input.astype(jnp.int32).reshape(NUM_TERMS, NUM_BAGS // CHUNK, CHUNK)
    mesh = plsc.VectorSubcoreMesh(core_axis_name="c", subcore_axis_name="s")
    run = pl.kernel(
        _bag_sum_body,
        out_type=jax.ShapeDtypeStruct((NUM_BAGS, EMB_DIM), jnp.float32),
        mesh=mesh,
        compiler_params=pltpu.CompilerParams(use_tc_tiling_on_sc=False),
        scratch_types=(
            [pltpu.VMEM((NUM_TERMS, CHUNKS_PER_TERM, CHUNK), jnp.int32),
             pltpu.VMEM((BAGS_PER_WORKER, EMB_DIM), jnp.float32)]
            + [pltpu.VMEM((CHUNK, EMB_DIM), jnp.float32)] * NBUF
            + [pltpu.SemaphoreType.DMA] * NBUF
        ),
    )
    return run(idx, weight)


# speedup vs baseline: 1.1895x; 1.1895x over previous
import jax
import jax.numpy as jnp
from jax import lax
from jax.experimental import pallas as pl
from jax.experimental.pallas import tpu as pltpu
from jax.experimental.pallas import tpu_sc as plsc

VOCAB_ROWS = 1_000_000
EMB_DIM = 32
NUM_TERMS = 50
NUM_BAGS = 16384
NUM_CORES = 2
NUM_SUBCORES = 16
NUM_WORKERS = NUM_CORES * NUM_SUBCORES
BAGS_PER_WORKER = NUM_BAGS // NUM_WORKERS
CHUNK = 128
CHUNKS_PER_TERM = BAGS_PER_WORKER // CHUNK


NBUF = 8
NUM_CHUNKS = NUM_TERMS * CHUNKS_PER_TERM


BATCH = 16


def _accumulate(acc, buf, c):
    for r0 in range(0, CHUNK, BATCH):
        xs = [buf[r, pl.ds(h, 16)]
              for r in range(r0, r0 + BATCH) for h in (0, 16)]
        for i, r in enumerate(range(r0, r0 + BATCH)):
            b = c * CHUNK + r
            plsc.addupdate(acc.at[b, pl.ds(0, 16)], xs[2 * i])
            plsc.addupdate(acc.at[b, pl.ds(16, 16)], xs[2 * i + 1])


def _bag_sum_body(idx_hbm, weight_hbm, out_hbm, idx_v, acc, *ring):
    bufs = ring[:NBUF]
    sems = ring[NBUF:]
    wid = lax.axis_index("s") * NUM_CORES + lax.axis_index("c")
    base = wid * BAGS_PER_WORKER

    pltpu.sync_copy(idx_hbm.at[:, pl.ds(wid * CHUNKS_PER_TERM,
                                        CHUNKS_PER_TERM)], idx_v)

    zero = jnp.zeros((16,), jnp.float32)

    @pl.loop(0, BAGS_PER_WORKER)
    def _(i):
        acc[i, pl.ds(0, 16)] = zero
        acc[i, pl.ds(16, 16)] = zero

    def start(l, c, b):
        pltpu.async_copy(weight_hbm.at[idx_v.at[l, c]], bufs[b], sems[b])

    def wait(b):
        pltpu.make_async_copy(weight_hbm.at[pl.ds(0, CHUNK)], bufs[b],
                              sems[b]).wait()

    for b in range(NBUF):
        start(b // CHUNKS_PER_TERM, b % CHUNKS_PER_TERM, b)

    @pl.loop(0, NUM_CHUNKS, step=NBUF)
    def _(j):
        for b in range(NBUF):
            jj = j + b
            wait(b)
            _accumulate(acc, bufs[b], b % CHUNKS_PER_TERM)
            nxt = jj + NBUF

            @pl.when(nxt < NUM_CHUNKS)
            def _():
                start(nxt // CHUNKS_PER_TERM, b % CHUNKS_PER_TERM, b)

    pltpu.sync_copy(acc, out_hbm.at[pl.ds(base, BAGS_PER_WORKER)])


def kernel(input, weight):
    idx = input.astype(jnp.int32).reshape(NUM_TERMS, NUM_BAGS // CHUNK, CHUNK)
    mesh = plsc.VectorSubcoreMesh(core_axis_name="c", subcore_axis_name="s")
    run = pl.kernel(
        _bag_sum_body,
        out_type=jax.ShapeDtypeStruct((NUM_BAGS, EMB_DIM), jnp.float32),
        mesh=mesh,
        compiler_params=pltpu.CompilerParams(use_tc_tiling_on_sc=False),
        scratch_types=(
            [pltpu.VMEM((NUM_TERMS, CHUNKS_PER_TERM, CHUNK), jnp.int32),
             pltpu.VMEM((BAGS_PER_WORKER, EMB_DIM), jnp.float32)]
            + [pltpu.VMEM((CHUNK, EMB_DIM), jnp.float32)] * NBUF
            + [pltpu.SemaphoreType.DMA] * NBUF
        ),
    )
    return run(idx, weight)
